# initial kernel scaffold (unmeasured)
import jax
import jax.numpy as jnp
from jax import lax
from jax.experimental import pallas as pl
from jax.experimental.pallas import tpu as pltpu


def kernel(
    x,
):
    def body(*refs):
        pass

    out_shape = jax.ShapeDtypeStruct(..., jnp.float32)
    return pl.pallas_call(body, out_shape=out_shape)(...)


# baseline (device time: 181126 ns/iter reference)
import jax
import jax.numpy as jnp
from jax import lax
from jax.experimental import pallas as pl
from jax.experimental.pallas import tpu as pltpu

N_DEV = 8
M = 1024
N_TOT = 4096
N_CHUNK = N_TOT // N_DEV


def kernel(x):
    def body(x_ref, out_ref, comm_ref, send_sems, recv_sems):
        p = lax.axis_index("i")
        left = jnp.mod(p - 1, N_DEV)
        right = jnp.mod(p + 1, N_DEV)

        barrier_sem = pltpu.get_barrier_semaphore()
        pl.semaphore_signal(
            barrier_sem, inc=1, device_id=(left,),
            device_id_type=pl.DeviceIdType.MESH,
        )
        pl.semaphore_signal(
            barrier_sem, inc=1, device_id=(right,),
            device_id_type=pl.DeviceIdType.MESH,
        )
        pl.semaphore_wait(barrier_sem, 2)

        c0 = jnp.mod(p - 1, N_DEV)
        comm_ref[N_DEV - 1] = x_ref[0, :, pl.ds(c0 * N_CHUNK, N_CHUNK)]

        for s in range(N_DEV - 1):
            src_slot = N_DEV - 1 if s == 0 else s - 1
            rdma = pltpu.make_async_remote_copy(
                src_ref=comm_ref.at[src_slot],
                dst_ref=comm_ref.at[s],
                send_sem=send_sems.at[s],
                recv_sem=recv_sems.at[s],
                device_id=(right,),
                device_id_type=pl.DeviceIdType.MESH,
            )
            rdma.start()
            rdma.wait()

            c = jnp.mod(p - 2 - s, N_DEV)
            if s < N_DEV - 2:
                comm_ref[s] = comm_ref[s] + x_ref[0, :, pl.ds(c * N_CHUNK, N_CHUNK)]
            else:
                out_ref[:, :] = comm_ref[s] + x_ref[0, :, pl.ds(p * N_CHUNK, N_CHUNK)]

    return pl.pallas_call(
        body,
        out_shape=jax.ShapeDtypeStruct((M, N_CHUNK), jnp.float32),
        in_specs=[pl.BlockSpec(memory_space=pltpu.VMEM)],
        out_specs=pl.BlockSpec(memory_space=pltpu.VMEM),
        scratch_shapes=[
            pltpu.VMEM((N_DEV, M, N_CHUNK), jnp.float32),
            pltpu.SemaphoreType.DMA((N_DEV - 1,)),
            pltpu.SemaphoreType.DMA((N_DEV - 1,)),
        ],
        compiler_params=pltpu.CompilerParams(collective_id=0),
    )(x)


# device time: 106254 ns/iter; 1.7047x vs baseline; 1.7047x over previous
import jax
import jax.numpy as jnp
from jax import lax
from jax.experimental import pallas as pl
from jax.experimental.pallas import tpu as pltpu

N_DEV = 8
M = 1024
N_TOT = 4096
N_CHUNK = N_TOT // N_DEV
N_HALF = N_CHUNK // 2


def kernel(x):
    def body(x_ref, out_ref, cw_ref, ccw_ref, cw_ssem, cw_rsem, ccw_ssem, ccw_rsem):
        p = lax.axis_index("i")
        left = jnp.mod(p - 1, N_DEV)
        right = jnp.mod(p + 1, N_DEV)

        barrier_sem = pltpu.get_barrier_semaphore()
        pl.semaphore_signal(
            barrier_sem, inc=1, device_id=(left,),
            device_id_type=pl.DeviceIdType.MESH,
        )
        pl.semaphore_signal(
            barrier_sem, inc=1, device_id=(right,),
            device_id_type=pl.DeviceIdType.MESH,
        )
        pl.semaphore_wait(barrier_sem, 2)

        c_cw = jnp.mod(p - 1, N_DEV)
        c_ccw = jnp.mod(p + 1, N_DEV)
        cw_ref[N_DEV - 1] = x_ref[0, :, pl.ds(c_cw * N_CHUNK, N_HALF)]
        ccw_ref[N_DEV - 1] = x_ref[0, :, pl.ds(c_ccw * N_CHUNK + N_HALF, N_HALF)]

        for s in range(N_DEV - 1):
            src = N_DEV - 1 if s == 0 else s - 1
            rdma_cw = pltpu.make_async_remote_copy(
                src_ref=cw_ref.at[src],
                dst_ref=cw_ref.at[s],
                send_sem=cw_ssem.at[s],
                recv_sem=cw_rsem.at[s],
                device_id=(right,),
                device_id_type=pl.DeviceIdType.MESH,
            )
            rdma_ccw = pltpu.make_async_remote_copy(
                src_ref=ccw_ref.at[src],
                dst_ref=ccw_ref.at[s],
                send_sem=ccw_ssem.at[s],
                recv_sem=ccw_rsem.at[s],
                device_id=(left,),
                device_id_type=pl.DeviceIdType.MESH,
            )
            rdma_cw.start()
            rdma_ccw.start()

            c_cw = jnp.mod(p - 2 - s, N_DEV)
            c_ccw = jnp.mod(p + 2 + s, N_DEV)
            rdma_cw.wait()
            if s < N_DEV - 2:
                cw_ref[s] = cw_ref[s] + x_ref[0, :, pl.ds(c_cw * N_CHUNK, N_HALF)]
            else:
                out_ref[:, 0:N_HALF] = (
                    cw_ref[s] + x_ref[0, :, pl.ds(p * N_CHUNK, N_HALF)]
                )
            rdma_ccw.wait()
            if s < N_DEV - 2:
                ccw_ref[s] = ccw_ref[s] + x_ref[
                    0, :, pl.ds(c_ccw * N_CHUNK + N_HALF, N_HALF)
                ]
            else:
                out_ref[:, N_HALF:N_CHUNK] = (
                    ccw_ref[s] + x_ref[0, :, pl.ds(p * N_CHUNK + N_HALF, N_HALF)]
                )

    return pl.pallas_call(
        body,
        out_shape=jax.ShapeDtypeStruct((M, N_CHUNK), jnp.float32),
        in_specs=[pl.BlockSpec(memory_space=pltpu.VMEM)],
        out_specs=pl.BlockSpec(memory_space=pltpu.VMEM),
        scratch_shapes=[
            pltpu.VMEM((N_DEV, M, N_HALF), jnp.float32),
            pltpu.VMEM((N_DEV, M, N_HALF), jnp.float32),
            pltpu.SemaphoreType.DMA((N_DEV - 1,)),
            pltpu.SemaphoreType.DMA((N_DEV - 1,)),
            pltpu.SemaphoreType.DMA((N_DEV - 1,)),
            pltpu.SemaphoreType.DMA((N_DEV - 1,)),
        ],
        compiler_params=pltpu.CompilerParams(collective_id=0),
    )(x)


# device time: 92682 ns/iter; 1.9543x vs baseline; 1.1464x over previous
import jax
import jax.numpy as jnp
from jax import lax
from jax.experimental import pallas as pl
from jax.experimental.pallas import tpu as pltpu

N_DEV = 8
M = 1024
N_TOT = 4096
N_CHUNK = N_TOT // N_DEV
N_HALF = N_CHUNK // 2
N_SEG = 2
SEG_M = M // N_SEG


def kernel(x):
    def body(x_ref, out_ref, cw_ref, ccw_ref, cw_ssem, cw_rsem, ccw_ssem, ccw_rsem):
        p = lax.axis_index("i")
        left = jnp.mod(p - 1, N_DEV)
        right = jnp.mod(p + 1, N_DEV)

        barrier_sem = pltpu.get_barrier_semaphore()
        pl.semaphore_signal(
            barrier_sem, inc=1, device_id=(left,),
            device_id_type=pl.DeviceIdType.MESH,
        )
        pl.semaphore_signal(
            barrier_sem, inc=1, device_id=(right,),
            device_id_type=pl.DeviceIdType.MESH,
        )
        pl.semaphore_wait(barrier_sem, 2)

        def mk(d_ref, ssem, rsem, tgt, s, k):
            src = N_DEV - 1 if s == 0 else s - 1
            return pltpu.make_async_remote_copy(
                src_ref=d_ref.at[src, pl.ds(k * SEG_M, SEG_M)],
                dst_ref=d_ref.at[s, pl.ds(k * SEG_M, SEG_M)],
                send_sem=ssem.at[s, k],
                recv_sem=rsem.at[s, k],
                device_id=(tgt,),
                device_id_type=pl.DeviceIdType.MESH,
            )

        def mk_cw(s, k):
            return mk(cw_ref, cw_ssem, cw_rsem, right, s, k)

        def mk_ccw(s, k):
            return mk(ccw_ref, ccw_ssem, ccw_rsem, left, s, k)

        c_cw0 = jnp.mod(p - 1, N_DEV)
        c_ccw0 = jnp.mod(p + 1, N_DEV)
        cw_ref[N_DEV - 1] = x_ref[0, :, pl.ds(c_cw0 * N_CHUNK, N_HALF)]
        ccw_ref[N_DEV - 1] = x_ref[0, :, pl.ds(c_ccw0 * N_CHUNK + N_HALF, N_HALF)]
        for k in range(N_SEG):
            mk_cw(0, k).start()
            mk_ccw(0, k).start()

        for s in range(N_DEV - 1):
            c_cw = jnp.mod(p - 2 - s, N_DEV)
            c_ccw = jnp.mod(p + 2 + s, N_DEV)
            for k in range(N_SEG):
                rows = pl.ds(k * SEG_M, SEG_M)
                mk_cw(s, k).wait_recv()
                if s < N_DEV - 2:
                    cw_ref[s, rows] = cw_ref[s, rows] + x_ref[
                        0, rows, pl.ds(c_cw * N_CHUNK, N_HALF)
                    ]
                    mk_cw(s + 1, k).start()
                else:
                    out_ref[rows, 0:N_HALF] = cw_ref[s, rows] + x_ref[
                        0, rows, pl.ds(p * N_CHUNK, N_HALF)
                    ]
                mk_ccw(s, k).wait_recv()
                if s < N_DEV - 2:
                    ccw_ref[s, rows] = ccw_ref[s, rows] + x_ref[
                        0, rows, pl.ds(c_ccw * N_CHUNK + N_HALF, N_HALF)
                    ]
                    mk_ccw(s + 1, k).start()
                else:
                    out_ref[rows, N_HALF:N_CHUNK] = ccw_ref[s, rows] + x_ref[
                        0, rows, pl.ds(p * N_CHUNK + N_HALF, N_HALF)
                    ]

        for s in range(N_DEV - 1):
            for k in range(N_SEG):
                mk_cw(s, k).wait_send()
                mk_ccw(s, k).wait_send()

    return pl.pallas_call(
        body,
        out_shape=jax.ShapeDtypeStruct((M, N_CHUNK), jnp.float32),
        in_specs=[pl.BlockSpec(memory_space=pltpu.VMEM)],
        out_specs=pl.BlockSpec(memory_space=pltpu.VMEM),
        scratch_shapes=[
            pltpu.VMEM((N_DEV, M, N_HALF), jnp.float32),
            pltpu.VMEM((N_DEV, M, N_HALF), jnp.float32),
            pltpu.SemaphoreType.DMA((N_DEV - 1, N_SEG)),
            pltpu.SemaphoreType.DMA((N_DEV - 1, N_SEG)),
            pltpu.SemaphoreType.DMA((N_DEV - 1, N_SEG)),
            pltpu.SemaphoreType.DMA((N_DEV - 1, N_SEG)),
        ],
        compiler_params=pltpu.CompilerParams(collective_id=0),
    )(x)
